# baseline (speedup 1.0000x reference)
import jax
import jax.numpy as jnp
from jax import lax
from jax.experimental import pallas as pl
from jax.experimental.pallas import tpu as pltpu
from jax.experimental.pallas import tpu_sc as plsc

N_ATOMS = 100000
D = 128
ZMAX = 87

NC = 2
NS = 16
NW = NC * NS

CHUNK = 128
N_CHUNKS = (N_ATOMS + CHUNK - 1) // CHUNK
UNITS = (N_CHUNKS + NW - 1) // NW
IDX_PER_W = UNITS * CHUNK
NBUF = 5


def _table_body(emb_ref, ec_ref, w_ref, out_ref):
    out_ref[...] = emb_ref[...] + lax.dot_general(
        ec_ref[...], w_ref[...],
        dimension_numbers=(((1,), (1,)), ((), ())),
        preferred_element_type=jnp.float32,
    )


def _build_table(element_embedding, W, electron_config):
    return pl.pallas_call(
        _table_body,
        out_shape=jax.ShapeDtypeStruct((ZMAX, D), jnp.float32),
    )(element_embedding, electron_config, W)


def _gather_body(table_hbm, z_hbm, out_hbm, idx_v, rows_v, stage_v, table_sh,
                 gsem, wsem, isem):
    sid = lax.axis_index("s")
    wid = sid * NC + lax.axis_index("c")
    base_w = jnp.minimum(wid * IDX_PER_W, N_ATOMS - IDX_PER_W)
    idx_dma = pltpu.async_copy(z_hbm.at[pl.ds(base_w, IDX_PER_W)], idx_v,
                               isem)

    @pl.when(sid == 0)
    def _():
        pltpu.sync_copy(table_hbm, stage_v)
        pltpu.sync_copy(stage_v, table_sh)

    plsc.subcore_barrier()
    idx_dma.wait()

    def issue_gather(u, b):
        pltpu.async_copy(
            table_sh.at[idx_v.at[pl.ds(u * CHUNK, CHUNK)]],
            rows_v.at[b], gsem)

    def wait_gather(b):
        pltpu.make_async_copy(
            out_hbm.at[pl.ds(0, CHUNK)], rows_v.at[b], gsem).wait()

    def issue_write(u, b):
        pltpu.async_copy(
            rows_v.at[b], out_hbm.at[pl.ds(base_w + u * CHUNK, CHUNK)], wsem)

    def wait_write(b):
        pltpu.make_async_copy(
            rows_v.at[b], out_hbm.at[pl.ds(0, CHUNK)], wsem).wait()

    for b in range(NBUF):
        issue_gather(jnp.int32(b), b)

    def body(u, carry):
        b = lax.rem(u, NBUF)
        wait_gather(b)
        issue_write(u, b)

        @pl.when((u >= 1) & (u - 1 + NBUF < UNITS))
        def _():
            bp = lax.rem(u - 1, NBUF)
            wait_write(bp)
            issue_gather(u - 1 + NBUF, bp)

        return carry

    lax.fori_loop(0, UNITS, body, 0)
    for u in range(UNITS - NBUF, UNITS):
        wait_write(u % NBUF)


_gather = pl.kernel(
    _gather_body,
    out_type=jax.ShapeDtypeStruct((N_ATOMS, D), jnp.float32),
    mesh=plsc.VectorSubcoreMesh(core_axis_name="c", subcore_axis_name="s"),
    scratch_types=[
        pltpu.VMEM((IDX_PER_W,), jnp.int32),
        pltpu.VMEM((NBUF, CHUNK, D), jnp.float32),
        pltpu.VMEM((ZMAX, D), jnp.float32),
        pltpu.VMEM_SHARED((ZMAX, D), jnp.float32),
        pltpu.SemaphoreType.DMA,
        pltpu.SemaphoreType.DMA,
        pltpu.SemaphoreType.DMA,
    ],
)


def kernel(Z, element_embedding, W, electron_config):
    table = _build_table(element_embedding, W, electron_config)
    if Z.dtype != jnp.int32:
        Z = Z.astype(jnp.int32)
    return _gather(table, Z)


# speedup vs baseline: 1.0197x; 1.0197x over previous
import jax
import jax.numpy as jnp
from jax import lax
from jax.experimental import pallas as pl
from jax.experimental.pallas import tpu as pltpu
from jax.experimental.pallas import tpu_sc as plsc

N_ATOMS = 100000
D = 128
ZMAX = 87

NC = 2
NS = 16
NW = NC * NS

CHUNK = 128
N_CHUNKS = (N_ATOMS + CHUNK - 1) // CHUNK
UNITS = (N_CHUNKS + NW - 1) // NW
IDX_PER_W = UNITS * CHUNK
NBUF = 5


def _table_body(emb_ref, ec_ref, w_ref, out_ref):
    out_ref[...] = emb_ref[...] + lax.dot_general(
        ec_ref[...], w_ref[...],
        dimension_numbers=(((1,), (1,)), ((), ())),
        preferred_element_type=jnp.float32,
    )


def _build_table(element_embedding, W, electron_config):
    return pl.pallas_call(
        _table_body,
        out_shape=jax.ShapeDtypeStruct((ZMAX, D), jnp.float32),
    )(element_embedding, electron_config, W)


def _gather_body(table_hbm, z_hbm, out_hbm, idx_v, rows_v, stage_v, table_sh,
                 gsem, wsem, isem):
    sid = lax.axis_index("s")
    wid = sid * NC + lax.axis_index("c")
    base_w = jnp.minimum(wid * IDX_PER_W, N_ATOMS - IDX_PER_W)
    idx_dma = pltpu.async_copy(z_hbm.at[pl.ds(base_w, IDX_PER_W)], idx_v,
                               isem)

    @pl.when(sid < 10)
    def _():
        pltpu.sync_copy(table_hbm.at[pl.ds(sid * 8, 8)], stage_v)
        pltpu.sync_copy(stage_v, table_sh.at[pl.ds(sid * 8, 8)])

    @pl.when(sid == 10)
    def _():
        pltpu.sync_copy(table_hbm.at[pl.ds(80, 7)], stage_v.at[pl.ds(0, 7)])
        pltpu.sync_copy(stage_v.at[pl.ds(0, 7)], table_sh.at[pl.ds(80, 7)])

    plsc.subcore_barrier()
    idx_dma.wait()

    def issue_gather(u, b):
        pltpu.async_copy(
            table_sh.at[idx_v.at[pl.ds(u * CHUNK, CHUNK)]],
            rows_v.at[b], gsem)

    def wait_gather(b):
        pltpu.make_async_copy(
            out_hbm.at[pl.ds(0, CHUNK)], rows_v.at[b], gsem).wait()

    def issue_write(u, b):
        pltpu.async_copy(
            rows_v.at[b], out_hbm.at[pl.ds(base_w + u * CHUNK, CHUNK)], wsem)

    def wait_write(b):
        pltpu.make_async_copy(
            rows_v.at[b], out_hbm.at[pl.ds(0, CHUNK)], wsem).wait()

    for b in range(NBUF):
        issue_gather(jnp.int32(b), b)

    def body(u, carry):
        b = lax.rem(u, NBUF)
        wait_gather(b)
        issue_write(u, b)

        @pl.when((u >= 1) & (u - 1 + NBUF < UNITS))
        def _():
            bp = lax.rem(u - 1, NBUF)
            wait_write(bp)
            issue_gather(u - 1 + NBUF, bp)

        return carry

    lax.fori_loop(0, UNITS, body, 0)
    for u in range(UNITS - NBUF, UNITS):
        wait_write(u % NBUF)


_gather = pl.kernel(
    _gather_body,
    out_type=jax.ShapeDtypeStruct((N_ATOMS, D), jnp.float32),
    mesh=plsc.VectorSubcoreMesh(core_axis_name="c", subcore_axis_name="s"),
    scratch_types=[
        pltpu.VMEM((IDX_PER_W,), jnp.int32),
        pltpu.VMEM((NBUF, CHUNK, D), jnp.float32),
        pltpu.VMEM((8, D), jnp.float32),
        pltpu.VMEM_SHARED((ZMAX, D), jnp.float32),
        pltpu.SemaphoreType.DMA,
        pltpu.SemaphoreType.DMA,
        pltpu.SemaphoreType.DMA,
    ],
)


def kernel(Z, element_embedding, W, electron_config):
    table = _build_table(element_embedding, W, electron_config)
    if Z.dtype != jnp.int32:
        Z = Z.astype(jnp.int32)
    return _gather(table, Z)
